# Initial kernel scaffold; baseline (speedup 1.0000x reference)
import jax
import jax.numpy as jnp
from jax.experimental import pallas as pl


def kernel(x, codebook):
    raise NotImplementedError("write your pallas kernel here")


# speedup vs baseline: 1.0247x; 1.0247x over previous
import functools

import jax
import jax.numpy as jnp
from jax import lax
from jax.experimental import pallas as pl
from jax.experimental.pallas import tpu as pltpu
from jax.experimental.pallas import tpu_sc as plsc

N_ROWS = 8192
NB = 8192
D = 32

ROW_BLK = 512
COL_BLK = 2048
N_ROW_BLKS = N_ROWS // ROW_BLK
N_COL_BLKS = NB // COL_BLK


def _quantize_kernel(xf_ref, kw_ref, rn_ref, cn_ref, idx_ref, loss_ref):
    xb = xf_ref[...]
    rn = rn_ref[...]
    run_min = jnp.full((ROW_BLK, 1), jnp.inf, jnp.float32)
    run_loss = jnp.zeros((ROW_BLK, 1), jnp.float32)
    run_idx = jnp.zeros((ROW_BLK, 1), jnp.int32)
    for c in range(N_COL_BLKS):
        kw_c = kw_ref[:, c * COL_BLK:(c + 1) * COL_BLK]
        cn_c = cn_ref[:, c * COL_BLK:(c + 1) * COL_BLK]
        m = jnp.dot(xb, kw_c, preferred_element_type=jnp.float32)
        d = (rn - 2.0 * m) + cn_c
        cmin = jnp.min(d, axis=-1, keepdims=True)
        col = lax.broadcasted_iota(jnp.int32, (ROW_BLK, COL_BLK), 1)
        cidx = jnp.min(jnp.where(d == cmin, col, COL_BLK), axis=-1,
                       keepdims=True) + c * COL_BLK
        better = cmin < run_min
        run_min = jnp.where(
            better, cmin.astype(jnp.bfloat16).astype(jnp.float32), run_min)
        run_loss = jnp.where(better, cmin, run_loss)
        run_idx = jnp.where(better, cidx, run_idx)
    idx_ref[...] = run_idx.reshape(1, 1, ROW_BLK)

    @pl.when(pl.program_id(0) == 0)
    def _init():
        loss_ref[...] = jnp.zeros_like(loss_ref)

    loss_ref[...] += jnp.sum(run_loss).reshape(1, 1)


_quantize = pl.pallas_call(
    _quantize_kernel,
    grid=(N_ROW_BLKS,),
    in_specs=[
        pl.BlockSpec((ROW_BLK, D), lambda i: (i, 0)),
        pl.BlockSpec((D, NB), lambda i: (0, 0)),
        pl.BlockSpec((ROW_BLK, 1), lambda i: (i, 0)),
        pl.BlockSpec((1, NB), lambda i: (0, 0)),
    ],
    out_specs=[
        pl.BlockSpec((1, 1, ROW_BLK), lambda i: (i, 0, 0)),
        pl.BlockSpec((1, 1), lambda i: (0, 0)),
    ],
    out_shape=[
        jax.ShapeDtypeStruct((N_ROW_BLKS, 1, ROW_BLK), jnp.int32),
        jax.ShapeDtypeStruct((1, 1), jnp.float32),
    ],
    compiler_params=pltpu.CompilerParams(
        dimension_semantics=("arbitrary",),
    ),
)


_NC, _NS = 2, 16
_NW = _NC * _NS
_B_PER_W = N_ROWS // _NW
D_PAD = 128


@functools.lru_cache(maxsize=1)
def _make_sc_gather():
    mesh = plsc.VectorSubcoreMesh(core_axis_name="c", subcore_axis_name="s",
                                  num_cores=_NC, num_subcores=_NS)

    @functools.partial(
        pl.kernel,
        out_type=jax.ShapeDtypeStruct((N_ROWS, D_PAD), jnp.float32),
        mesh=mesh,
        scratch_types=[
            pltpu.VMEM((_B_PER_W,), jnp.int32),
            pltpu.VMEM((_B_PER_W, D_PAD), jnp.float32),
            pltpu.SemaphoreType.DMA,
        ],
    )
    def _sc_gather(table_hbm, idx_hbm, out_hbm, idx_v, rows_v, sem):
        wid = lax.axis_index("s") * _NC + lax.axis_index("c")
        base = wid * _B_PER_W
        pltpu.sync_copy(idx_hbm.at[pl.ds(base, _B_PER_W)], idx_v)
        pltpu.async_copy(table_hbm.at[idx_v], rows_v, sem).wait()
        pltpu.sync_copy(rows_v, out_hbm.at[pl.ds(base, _B_PER_W)])

    return _sc_gather


def kernel(x, codebook):
    bs, c, f, j = x.shape
    xf = jnp.transpose(x, (0, 2, 3, 1)).reshape(-1, c)
    k_w = codebook.T
    rn = jnp.sum(xf ** 2, axis=-1, keepdims=True)
    cn = jnp.sum(k_w ** 2, axis=0, keepdims=True)
    idx3, loss_sum = _quantize(xf, k_w, rn, cn)
    idx = idx3.reshape(N_ROWS)
    codebook_p = jnp.pad(codebook, ((0, 0), (0, D_PAD - D)))
    x_d = _make_sc_gather()(codebook_p, idx)[:, :D]
    commit_loss = loss_sum.reshape(()) / float(N_ROWS * D)
    x_st = xf + (x_d - xf)
    out = jnp.transpose(x_st.reshape(bs, f, j, c), (0, 3, 1, 2))
    return (out, commit_loss)


# speedup vs baseline: 1.1578x; 1.1299x over previous
import functools

import jax
import jax.numpy as jnp
from jax import lax
from jax.experimental import pallas as pl
from jax.experimental.pallas import tpu as pltpu
from jax.experimental.pallas import tpu_sc as plsc

N_ROWS = 8192
NB = 8192
D = 32

ROW_BLK = 512
COL_BLK = 2048
N_ROW_BLKS = N_ROWS // ROW_BLK
N_COL_BLKS = NB // COL_BLK


def _quantize_kernel(xf_ref, kw2_ref, rn_ref, cn_ref, idx_ref, loss_ref):
    xb = xf_ref[...]
    rn = rn_ref[...]
    colf = lax.broadcasted_iota(jnp.int32, (ROW_BLK, COL_BLK), 1).astype(
        jnp.float32)
    run_min = jnp.full((ROW_BLK, 1), jnp.inf, jnp.float32)
    run_loss = jnp.zeros((ROW_BLK, 1), jnp.float32)
    run_idx = jnp.zeros((ROW_BLK, 1), jnp.float32)
    for c in range(N_COL_BLKS):
        kw_c = kw2_ref[:, c * COL_BLK:(c + 1) * COL_BLK]
        cn_c = cn_ref[:, c * COL_BLK:(c + 1) * COL_BLK]
        m2 = jnp.dot(xb, kw_c, preferred_element_type=jnp.float32)
        d = (rn + m2) + cn_c
        cmin = jnp.min(d, axis=-1, keepdims=True)
        cidx = jnp.min(jnp.where(d == cmin, colf, float(COL_BLK)), axis=-1,
                       keepdims=True)
        better = cmin < run_min
        run_min = jnp.where(
            better, cmin.astype(jnp.bfloat16).astype(jnp.float32), run_min)
        run_loss = jnp.where(better, cmin, run_loss)
        run_idx = jnp.where(better, cidx + float(c * COL_BLK), run_idx)
    idx_ref[...] = run_idx.astype(jnp.int32).reshape(1, 1, ROW_BLK)

    @pl.when(pl.program_id(0) == 0)
    def _init():
        loss_ref[...] = jnp.zeros_like(loss_ref)

    loss_ref[...] += jnp.sum(run_loss).reshape(1, 1)


_quantize = pl.pallas_call(
    _quantize_kernel,
    grid=(N_ROW_BLKS,),
    in_specs=[
        pl.BlockSpec((ROW_BLK, D), lambda i: (i, 0)),
        pl.BlockSpec((D, NB), lambda i: (0, 0)),
        pl.BlockSpec((ROW_BLK, 1), lambda i: (i, 0)),
        pl.BlockSpec((1, NB), lambda i: (0, 0)),
    ],
    out_specs=[
        pl.BlockSpec((1, 1, ROW_BLK), lambda i: (i, 0, 0)),
        pl.BlockSpec((1, 1), lambda i: (0, 0)),
    ],
    out_shape=[
        jax.ShapeDtypeStruct((N_ROW_BLKS, 1, ROW_BLK), jnp.int32),
        jax.ShapeDtypeStruct((1, 1), jnp.float32),
    ],
    compiler_params=pltpu.CompilerParams(
        dimension_semantics=("arbitrary",),
    ),
)


_NC, _NS = 2, 16
_NW = _NC * _NS
_B_PER_W = N_ROWS // _NW
D_PAD = 128


@functools.lru_cache(maxsize=1)
def _make_sc_gather():
    mesh = plsc.VectorSubcoreMesh(core_axis_name="c", subcore_axis_name="s",
                                  num_cores=_NC, num_subcores=_NS)

    @functools.partial(
        pl.kernel,
        out_type=jax.ShapeDtypeStruct((N_ROWS, D_PAD), jnp.float32),
        mesh=mesh,
        scratch_types=[
            pltpu.VMEM((_B_PER_W,), jnp.int32),
            pltpu.VMEM((_B_PER_W, D_PAD), jnp.float32),
            pltpu.SemaphoreType.DMA,
        ],
    )
    def _sc_gather(table_hbm, idx_hbm, out_hbm, idx_v, rows_v, sem):
        wid = lax.axis_index("s") * _NC + lax.axis_index("c")
        base = wid * _B_PER_W
        pltpu.sync_copy(idx_hbm.at[pl.ds(base, _B_PER_W)], idx_v)
        pltpu.async_copy(table_hbm.at[idx_v], rows_v, sem).wait()
        pltpu.sync_copy(rows_v, out_hbm.at[pl.ds(base, _B_PER_W)])

    return _sc_gather


def kernel(x, codebook):
    bs, c, f, j = x.shape
    xf = jnp.transpose(x, (0, 2, 3, 1)).reshape(-1, c)
    k_w = codebook.T
    rn = jnp.sum(xf ** 2, axis=-1, keepdims=True)
    cn = jnp.sum(k_w ** 2, axis=0, keepdims=True)
    kw2 = k_w * (-2.0)
    idx3, loss_sum = _quantize(xf, kw2, rn, cn)
    idx = idx3.reshape(N_ROWS)
    codebook_p = jnp.pad(codebook, ((0, 0), (0, D_PAD - D)))
    x_d = _make_sc_gather()(codebook_p, idx)[:, :D]
    commit_loss = loss_sum.reshape(()) / float(N_ROWS * D)
    x_st = xf + (x_d - xf)
    out = jnp.transpose(x_st.reshape(bs, f, j, c), (0, 3, 1, 2))
    return (out, commit_loss)


# speedup vs baseline: 1.1939x; 1.0312x over previous
import functools

import jax
import jax.numpy as jnp
from jax import lax
from jax.experimental import pallas as pl
from jax.experimental.pallas import tpu as pltpu
from jax.experimental.pallas import tpu_sc as plsc

N_ROWS = 8192
NB = 8192
D = 32

ROW_BLK = 512
COL_BLK = 2048
N_ROW_BLKS = N_ROWS // ROW_BLK
N_COL_BLKS = NB // COL_BLK


def _quantize_kernel(xf_ref, kw_ref, rn_ref, cn_ref, idx_ref, loss_ref):
    xb = xf_ref[...]
    rn = rn_ref[...]
    colf = lax.broadcasted_iota(jnp.int32, (ROW_BLK, COL_BLK), 1).astype(
        jnp.float32)
    run_min = jnp.full((ROW_BLK, 1), jnp.inf, jnp.float32)
    run_loss = jnp.zeros((ROW_BLK, 1), jnp.float32)
    run_idx = jnp.zeros((ROW_BLK, 1), jnp.float32)
    for c in range(N_COL_BLKS):
        kw_c = kw_ref[:, c * COL_BLK:(c + 1) * COL_BLK] * (-2.0)
        cn_c = cn_ref[:, c * COL_BLK:(c + 1) * COL_BLK]
        m2 = jnp.dot(xb, kw_c, preferred_element_type=jnp.float32)
        d = (rn + m2) + cn_c
        cmin = jnp.min(d, axis=-1, keepdims=True)
        cidx = jnp.min(jnp.where(d == cmin, colf, float(COL_BLK)), axis=-1,
                       keepdims=True)
        better = cmin < run_min
        run_min = jnp.where(
            better, cmin.astype(jnp.bfloat16).astype(jnp.float32), run_min)
        run_loss = jnp.where(better, cmin, run_loss)
        run_idx = jnp.where(better, cidx + float(c * COL_BLK), run_idx)
    idx_ref[...] = run_idx.astype(jnp.int32).reshape(1, 1, ROW_BLK)

    @pl.when(pl.program_id(0) == 0)
    def _init():
        loss_ref[...] = jnp.zeros_like(loss_ref)

    loss_ref[...] += jnp.sum(run_loss).reshape(1, 1)


_quantize = pl.pallas_call(
    _quantize_kernel,
    grid=(N_ROW_BLKS,),
    in_specs=[
        pl.BlockSpec((ROW_BLK, D), lambda i: (i, 0)),
        pl.BlockSpec((D, NB), lambda i: (0, 0)),
        pl.BlockSpec((ROW_BLK, 1), lambda i: (i, 0)),
        pl.BlockSpec((1, NB), lambda i: (0, 0)),
    ],
    out_specs=[
        pl.BlockSpec((1, 1, ROW_BLK), lambda i: (i, 0, 0)),
        pl.BlockSpec((1, 1), lambda i: (0, 0)),
    ],
    out_shape=[
        jax.ShapeDtypeStruct((N_ROW_BLKS, 1, ROW_BLK), jnp.int32),
        jax.ShapeDtypeStruct((1, 1), jnp.float32),
    ],
    compiler_params=pltpu.CompilerParams(
        dimension_semantics=("arbitrary",),
    ),
)


_NC, _NS = 2, 16
_NW = _NC * _NS
_B_PER_W = N_ROWS // _NW
D_PAD = 128


@functools.lru_cache(maxsize=1)
def _make_sc_gather():
    mesh = plsc.VectorSubcoreMesh(core_axis_name="c", subcore_axis_name="s",
                                  num_cores=_NC, num_subcores=_NS)

    @functools.partial(
        pl.kernel,
        out_type=jax.ShapeDtypeStruct((N_ROWS, D_PAD), jnp.float32),
        mesh=mesh,
        scratch_types=[
            pltpu.VMEM((_B_PER_W,), jnp.int32),
            pltpu.VMEM((_B_PER_W, D_PAD), jnp.float32),
            pltpu.SemaphoreType.DMA,
        ],
    )
    def _sc_gather(table_hbm, idx_hbm, out_hbm, idx_v, rows_v, sem):
        wid = lax.axis_index("s") * _NC + lax.axis_index("c")
        base = wid * _B_PER_W
        pltpu.sync_copy(idx_hbm.at[pl.ds(base, _B_PER_W)], idx_v)
        pltpu.async_copy(table_hbm.at[idx_v], rows_v, sem).wait()
        pltpu.sync_copy(rows_v, out_hbm.at[pl.ds(base, _B_PER_W)])

    return _sc_gather


BS, C, F, J = 8, 32, 32, 32
ROWS_PER_B = F * J


def _finalize_kernel(x_ref, xd_ref, out_ref):
    xc = x_ref[...].reshape(C, ROWS_PER_B)
    xdt = jnp.transpose(xd_ref[:, :D], (1, 0))
    out_ref[...] = (xc + (xdt - xc)).reshape(1, C, F, J)


_finalize = pl.pallas_call(
    _finalize_kernel,
    grid=(BS,),
    in_specs=[
        pl.BlockSpec((1, C, F, J), lambda i: (i, 0, 0, 0)),
        pl.BlockSpec((ROWS_PER_B, D_PAD), lambda i: (i, 0)),
    ],
    out_specs=pl.BlockSpec((1, C, F, J), lambda i: (i, 0, 0, 0)),
    out_shape=jax.ShapeDtypeStruct((BS, C, F, J), jnp.float32),
    compiler_params=pltpu.CompilerParams(
        dimension_semantics=("arbitrary",),
    ),
)


def kernel(x, codebook):
    bs, c, f, j = x.shape
    xf = jnp.transpose(x, (0, 2, 3, 1)).reshape(-1, c)
    k_w = codebook.T
    rn = jnp.sum(xf ** 2, axis=-1, keepdims=True)
    cn = jnp.sum(k_w ** 2, axis=0, keepdims=True)
    idx3, loss_sum = _quantize(xf, k_w, rn, cn)
    idx = idx3.reshape(N_ROWS)
    codebook_p = jnp.pad(codebook, ((0, 0), (0, D_PAD - D)))
    xd_p = _make_sc_gather()(codebook_p, idx)
    commit_loss = loss_sum.reshape(()) / float(N_ROWS * D)
    out = _finalize(x, xd_p)
    return (out, commit_loss)


# speedup vs baseline: 1.2012x; 1.0061x over previous
import functools

import jax
import jax.numpy as jnp
from jax import lax
from jax.experimental import pallas as pl
from jax.experimental.pallas import tpu as pltpu
from jax.experimental.pallas import tpu_sc as plsc

N_ROWS = 8192
NB = 8192
D = 32

ROW_BLK = 512
COL_BLK = 2048
N_ROW_BLKS = N_ROWS // ROW_BLK
N_COL_BLKS = NB // COL_BLK


F_SUB = 16


def _quantize_kernel(x_ref, kw_ref, rn_ref, cn_ref, idx_ref, loss_ref):
    xb = jnp.transpose(x_ref[...].reshape(D, ROW_BLK), (1, 0))
    rn = rn_ref[...]
    colf = lax.broadcasted_iota(jnp.int32, (ROW_BLK, COL_BLK), 1).astype(
        jnp.float32)
    run_min = jnp.full((ROW_BLK, 1), jnp.inf, jnp.float32)
    run_loss = jnp.zeros((ROW_BLK, 1), jnp.float32)
    run_idx = jnp.zeros((ROW_BLK, 1), jnp.float32)
    for c in range(N_COL_BLKS):
        kw_c = kw_ref[:, c * COL_BLK:(c + 1) * COL_BLK] * (-2.0)
        cn_c = cn_ref[:, c * COL_BLK:(c + 1) * COL_BLK]
        m2 = jnp.dot(xb, kw_c, preferred_element_type=jnp.float32)
        d = (rn + m2) + cn_c
        cmin = jnp.min(d, axis=-1, keepdims=True)
        cidx = jnp.min(jnp.where(d == cmin, colf, float(COL_BLK)), axis=-1,
                       keepdims=True)
        better = cmin < run_min
        run_min = jnp.where(
            better, cmin.astype(jnp.bfloat16).astype(jnp.float32), run_min)
        run_loss = jnp.where(better, cmin, run_loss)
        run_idx = jnp.where(better, cidx + float(c * COL_BLK), run_idx)
    idx_ref[...] = run_idx.astype(jnp.int32).reshape(1, 1, ROW_BLK)

    @pl.when(pl.program_id(0) == 0)
    def _init():
        loss_ref[...] = jnp.zeros_like(loss_ref)

    loss_ref[...] += jnp.sum(run_loss).reshape(1, 1)


_quantize = pl.pallas_call(
    _quantize_kernel,
    grid=(N_ROW_BLKS,),
    in_specs=[
        pl.BlockSpec((1, D, F_SUB, 32), lambda i: (i // 2, 0, i % 2, 0)),
        pl.BlockSpec((D, NB), lambda i: (0, 0)),
        pl.BlockSpec((ROW_BLK, 1), lambda i: (i, 0)),
        pl.BlockSpec((1, NB), lambda i: (0, 0)),
    ],
    out_specs=[
        pl.BlockSpec((1, 1, ROW_BLK), lambda i: (i, 0, 0)),
        pl.BlockSpec((1, 1), lambda i: (0, 0)),
    ],
    out_shape=[
        jax.ShapeDtypeStruct((N_ROW_BLKS, 1, ROW_BLK), jnp.int32),
        jax.ShapeDtypeStruct((1, 1), jnp.float32),
    ],
    compiler_params=pltpu.CompilerParams(
        dimension_semantics=("arbitrary",),
    ),
)


_NC, _NS = 2, 16
_NW = _NC * _NS
_B_PER_W = N_ROWS // _NW
D_PAD = 128


@functools.lru_cache(maxsize=1)
def _make_sc_gather():
    mesh = plsc.VectorSubcoreMesh(core_axis_name="c", subcore_axis_name="s",
                                  num_cores=_NC, num_subcores=_NS)

    @functools.partial(
        pl.kernel,
        out_type=jax.ShapeDtypeStruct((N_ROWS, D_PAD), jnp.float32),
        mesh=mesh,
        scratch_types=[
            pltpu.VMEM((_B_PER_W,), jnp.int32),
            pltpu.VMEM((_B_PER_W, D_PAD), jnp.float32),
            pltpu.SemaphoreType.DMA,
        ],
    )
    def _sc_gather(table_hbm, idx_hbm, out_hbm, idx_v, rows_v, sem):
        wid = lax.axis_index("s") * _NC + lax.axis_index("c")
        base = wid * _B_PER_W
        pltpu.sync_copy(idx_hbm.at[pl.ds(base, _B_PER_W)], idx_v)
        pltpu.async_copy(table_hbm.at[idx_v], rows_v, sem).wait()
        pltpu.sync_copy(rows_v, out_hbm.at[pl.ds(base, _B_PER_W)])

    return _sc_gather


BS, C, F, J = 8, 32, 32, 32
ROWS_PER_B = F * J


def _finalize_kernel(x_ref, xd_ref, out_ref):
    xc = x_ref[...].reshape(C, ROWS_PER_B)
    xdt = jnp.transpose(xd_ref[:, :D], (1, 0))
    out_ref[...] = (xc + (xdt - xc)).reshape(1, C, F, J)


_finalize = pl.pallas_call(
    _finalize_kernel,
    grid=(BS,),
    in_specs=[
        pl.BlockSpec((1, C, F, J), lambda i: (i, 0, 0, 0)),
        pl.BlockSpec((ROWS_PER_B, D_PAD), lambda i: (i, 0)),
    ],
    out_specs=pl.BlockSpec((1, C, F, J), lambda i: (i, 0, 0, 0)),
    out_shape=jax.ShapeDtypeStruct((BS, C, F, J), jnp.float32),
    compiler_params=pltpu.CompilerParams(
        dimension_semantics=("arbitrary",),
    ),
)


def kernel(x, codebook):
    bs, c, f, j = x.shape
    xf = jnp.transpose(x, (0, 2, 3, 1)).reshape(-1, c)
    k_w = codebook.T
    rn = jnp.sum(xf ** 2, axis=-1, keepdims=True)
    cn = jnp.sum(k_w ** 2, axis=0, keepdims=True)
    idx3, loss_sum = _quantize(x, k_w, rn, cn)
    idx = idx3.reshape(N_ROWS)
    codebook_p = jnp.pad(codebook, ((0, 0), (0, D_PAD - D)))
    xd_p = _make_sc_gather()(codebook_p, idx)
    commit_loss = loss_sum.reshape(()) / float(N_ROWS * D)
    out = _finalize(x, xd_p)
    return (out, commit_loss)


# speedup vs baseline: 1.2027x; 1.0012x over previous
import functools

import jax
import jax.numpy as jnp
from jax import lax
from jax.experimental import pallas as pl
from jax.experimental.pallas import tpu as pltpu
from jax.experimental.pallas import tpu_sc as plsc

N_ROWS = 8192
NB = 8192
D = 32

ROW_BLK = 1024
COL_BLK = 2048
N_ROW_BLKS = N_ROWS // ROW_BLK
N_COL_BLKS = NB // COL_BLK


F_SUB = 32


def _quantize_kernel(x_ref, kw_ref, rn_ref, cn_ref, idx_ref, loss_ref):
    xb = jnp.transpose(x_ref[...].reshape(D, ROW_BLK), (1, 0))
    rn = rn_ref[...]
    colf = lax.broadcasted_iota(jnp.int32, (ROW_BLK, COL_BLK), 1).astype(
        jnp.float32)
    run_min = jnp.full((ROW_BLK, 1), jnp.inf, jnp.float32)
    run_loss = jnp.zeros((ROW_BLK, 1), jnp.float32)
    run_idx = jnp.zeros((ROW_BLK, 1), jnp.float32)
    for c in range(N_COL_BLKS):
        kw_c = kw_ref[c * COL_BLK:(c + 1) * COL_BLK, :] * (-2.0)
        cn_c = cn_ref[:, c * COL_BLK:(c + 1) * COL_BLK]
        m2 = lax.dot_general(xb, kw_c, (((1,), (1,)), ((), ())),
                             preferred_element_type=jnp.float32)
        d = (rn + m2) + cn_c
        cmin = jnp.min(d, axis=-1, keepdims=True)
        cidx = jnp.min(jnp.where(d == cmin, colf, float(COL_BLK)), axis=-1,
                       keepdims=True)
        better = cmin < run_min
        run_min = jnp.where(
            better, cmin.astype(jnp.bfloat16).astype(jnp.float32), run_min)
        run_loss = jnp.where(better, cmin, run_loss)
        run_idx = jnp.where(better, cidx + float(c * COL_BLK), run_idx)
    idx_ref[...] = run_idx.astype(jnp.int32).reshape(1, 1, ROW_BLK)

    @pl.when(pl.program_id(0) == 0)
    def _init():
        loss_ref[...] = jnp.zeros_like(loss_ref)

    loss_ref[...] += jnp.sum(run_loss).reshape(1, 1)


_quantize = pl.pallas_call(
    _quantize_kernel,
    grid=(N_ROW_BLKS,),
    in_specs=[
        pl.BlockSpec((1, D, F_SUB, 32), lambda i: (i, 0, 0, 0)),
        pl.BlockSpec((NB, D), lambda i: (0, 0)),
        pl.BlockSpec((ROW_BLK, 1), lambda i: (i, 0)),
        pl.BlockSpec((1, NB), lambda i: (0, 0)),
    ],
    out_specs=[
        pl.BlockSpec((1, 1, ROW_BLK), lambda i: (i, 0, 0)),
        pl.BlockSpec((1, 1), lambda i: (0, 0)),
    ],
    out_shape=[
        jax.ShapeDtypeStruct((N_ROW_BLKS, 1, ROW_BLK), jnp.int32),
        jax.ShapeDtypeStruct((1, 1), jnp.float32),
    ],
    compiler_params=pltpu.CompilerParams(
        dimension_semantics=("arbitrary",),
    ),
)


_NC, _NS = 2, 16
_NW = _NC * _NS
_B_PER_W = N_ROWS // _NW
D_PAD = 128


@functools.lru_cache(maxsize=1)
def _make_sc_gather():
    mesh = plsc.VectorSubcoreMesh(core_axis_name="c", subcore_axis_name="s",
                                  num_cores=_NC, num_subcores=_NS)

    @functools.partial(
        pl.kernel,
        out_type=jax.ShapeDtypeStruct((N_ROWS, D_PAD), jnp.float32),
        mesh=mesh,
        scratch_types=[
            pltpu.VMEM((_B_PER_W,), jnp.int32),
            pltpu.VMEM((_B_PER_W, D_PAD), jnp.float32),
            pltpu.SemaphoreType.DMA,
        ],
    )
    def _sc_gather(table_hbm, idx_hbm, out_hbm, idx_v, rows_v, sem):
        wid = lax.axis_index("s") * _NC + lax.axis_index("c")
        base = wid * _B_PER_W
        pltpu.sync_copy(idx_hbm.at[pl.ds(base, _B_PER_W)], idx_v)
        pltpu.async_copy(table_hbm.at[idx_v], rows_v, sem).wait()
        pltpu.sync_copy(rows_v, out_hbm.at[pl.ds(base, _B_PER_W)])

    return _sc_gather


BS, C, F, J = 8, 32, 32, 32
ROWS_PER_B = F * J


def _finalize_kernel(x_ref, xd_ref, out_ref):
    xc = x_ref[...].reshape(C, ROWS_PER_B)
    xdt = jnp.transpose(xd_ref[:, :D], (1, 0))
    out_ref[...] = (xc + (xdt - xc)).reshape(1, C, F, J)


_finalize = pl.pallas_call(
    _finalize_kernel,
    grid=(BS,),
    in_specs=[
        pl.BlockSpec((1, C, F, J), lambda i: (i, 0, 0, 0)),
        pl.BlockSpec((ROWS_PER_B, D_PAD), lambda i: (i, 0)),
    ],
    out_specs=pl.BlockSpec((1, C, F, J), lambda i: (i, 0, 0, 0)),
    out_shape=jax.ShapeDtypeStruct((BS, C, F, J), jnp.float32),
    compiler_params=pltpu.CompilerParams(
        dimension_semantics=("arbitrary",),
    ),
)


def kernel(x, codebook):
    bs, c, f, j = x.shape
    xf = jnp.transpose(x, (0, 2, 3, 1)).reshape(-1, c)
    k_w = codebook.T
    rn = jnp.sum(xf ** 2, axis=-1, keepdims=True)
    cn = jnp.sum(k_w ** 2, axis=0, keepdims=True)
    idx3, loss_sum = _quantize(x, codebook, rn, cn)
    idx = idx3.reshape(N_ROWS)
    codebook_p = jnp.pad(codebook, ((0, 0), (0, D_PAD - D)))
    xd_p = _make_sc_gather()(codebook_p, idx)
    commit_loss = loss_sum.reshape(()) / float(N_ROWS * D)
    out = _finalize(x, xd_p)
    return (out, commit_loss)


# speedup vs baseline: 1.2125x; 1.0082x over previous
import functools

import jax
import jax.numpy as jnp
from jax import lax
from jax.experimental import pallas as pl
from jax.experimental.pallas import tpu as pltpu
from jax.experimental.pallas import tpu_sc as plsc

N_ROWS = 8192
NB = 8192
D = 32

ROW_BLK = 1024
COL_BLK = 2048
N_ROW_BLKS = N_ROWS // ROW_BLK
N_COL_BLKS = NB // COL_BLK


F_SUB = 32


def _quantize_kernel(x_ref, kw_ref, rn_ref, cn_ref, idx_ref, loss_ref):
    xb = jnp.transpose(x_ref[...].reshape(D, ROW_BLK), (1, 0))
    rn = rn_ref[...]
    colf = lax.broadcasted_iota(jnp.int32, (ROW_BLK, COL_BLK), 1).astype(
        jnp.float32)
    run_min = jnp.full((ROW_BLK, 1), jnp.inf, jnp.float32)
    run_loss = jnp.zeros((ROW_BLK, 1), jnp.float32)
    run_idx = jnp.zeros((ROW_BLK, 1), jnp.float32)
    for c in range(N_COL_BLKS):
        kw_c = kw_ref[c * COL_BLK:(c + 1) * COL_BLK, :] * (-2.0)
        cn_c = cn_ref[:, c * COL_BLK:(c + 1) * COL_BLK]
        m2 = lax.dot_general(xb, kw_c, (((1,), (1,)), ((), ())),
                             preferred_element_type=jnp.float32)
        d = (rn + m2) + cn_c
        cmin = jnp.min(d, axis=-1, keepdims=True)
        cidx = jnp.min(jnp.where(d == cmin, colf, float(COL_BLK)), axis=-1,
                       keepdims=True)
        better = cmin < run_min
        run_min = jnp.where(
            better, cmin.astype(jnp.bfloat16).astype(jnp.float32), run_min)
        run_loss = jnp.where(better, cmin, run_loss)
        run_idx = jnp.where(better, cidx + float(c * COL_BLK), run_idx)
    idx_ref[...] = run_idx.astype(jnp.int32).reshape(1, 1, ROW_BLK)

    @pl.when(pl.program_id(0) == 0)
    def _init():
        loss_ref[...] = jnp.zeros_like(loss_ref)

    loss_ref[...] += jnp.sum(run_loss).reshape(1, 1)

    @pl.when(pl.program_id(0) == N_ROW_BLKS - 1)
    def _scale():
        loss_ref[...] = loss_ref[...] * (1.0 / float(N_ROWS * D))


_quantize = pl.pallas_call(
    _quantize_kernel,
    grid=(N_ROW_BLKS,),
    in_specs=[
        pl.BlockSpec((1, D, F_SUB, 32), lambda i: (i, 0, 0, 0)),
        pl.BlockSpec((NB, D), lambda i: (0, 0)),
        pl.BlockSpec((ROW_BLK, 1), lambda i: (i, 0)),
        pl.BlockSpec((1, NB), lambda i: (0, 0)),
    ],
    out_specs=[
        pl.BlockSpec((1, 1, ROW_BLK), lambda i: (i, 0, 0)),
        pl.BlockSpec((1, 1), lambda i: (0, 0)),
    ],
    out_shape=[
        jax.ShapeDtypeStruct((N_ROW_BLKS, 1, ROW_BLK), jnp.int32),
        jax.ShapeDtypeStruct((1, 1), jnp.float32),
    ],
    compiler_params=pltpu.CompilerParams(
        dimension_semantics=("arbitrary",),
    ),
)


_NC, _NS = 2, 16
_NW = _NC * _NS
_B_PER_W = N_ROWS // _NW
D_PAD = 128


@functools.lru_cache(maxsize=1)
def _make_sc_gather():
    mesh = plsc.VectorSubcoreMesh(core_axis_name="c", subcore_axis_name="s",
                                  num_cores=_NC, num_subcores=_NS)

    @functools.partial(
        pl.kernel,
        out_type=jax.ShapeDtypeStruct((N_ROWS, D_PAD), jnp.float32),
        mesh=mesh,
        scratch_types=[
            pltpu.VMEM((_B_PER_W,), jnp.int32),
            pltpu.VMEM((_B_PER_W, D_PAD), jnp.float32),
            pltpu.SemaphoreType.DMA,
        ],
    )
    def _sc_gather(table_hbm, idx_hbm, out_hbm, idx_v, rows_v, sem):
        wid = lax.axis_index("s") * _NC + lax.axis_index("c")
        base = wid * _B_PER_W
        pltpu.sync_copy(idx_hbm.at[pl.ds(base, _B_PER_W)], idx_v)
        pltpu.async_copy(table_hbm.at[idx_v], rows_v, sem).wait()
        pltpu.sync_copy(rows_v, out_hbm.at[pl.ds(base, _B_PER_W)])

    return _sc_gather


BS, C, F, J = 8, 32, 32, 32
ROWS_PER_B = F * J


def _finalize_kernel(x_ref, xd_ref, out_ref):
    xc = x_ref[...].reshape(C, ROWS_PER_B)
    xdt = jnp.transpose(xd_ref[:, :D], (1, 0))
    out_ref[...] = (xc + (xdt - xc)).reshape(1, C, F, J)


_finalize = pl.pallas_call(
    _finalize_kernel,
    grid=(BS,),
    in_specs=[
        pl.BlockSpec((1, C, F, J), lambda i: (i, 0, 0, 0)),
        pl.BlockSpec((ROWS_PER_B, D_PAD), lambda i: (i, 0)),
    ],
    out_specs=pl.BlockSpec((1, C, F, J), lambda i: (i, 0, 0, 0)),
    out_shape=jax.ShapeDtypeStruct((BS, C, F, J), jnp.float32),
    compiler_params=pltpu.CompilerParams(
        dimension_semantics=("arbitrary",),
    ),
)


def kernel(x, codebook):
    bs, c, f, j = x.shape
    xf = jnp.transpose(x, (0, 2, 3, 1)).reshape(-1, c)
    k_w = codebook.T
    rn = jnp.sum(xf ** 2, axis=-1, keepdims=True)
    cn = jnp.sum(k_w ** 2, axis=0, keepdims=True)
    idx3, loss_sum = _quantize(x, codebook, rn, cn)
    idx = idx3.reshape(N_ROWS)
    codebook_p = jnp.pad(codebook, ((0, 0), (0, D_PAD - D)))
    xd_p = _make_sc_gather()(codebook_p, idx)
    commit_loss = loss_sum.reshape(())
    out = _finalize(x, xd_p)
    return (out, commit_loss)
